# baseline (device time: 43446 ns/iter reference)
import jax
import jax.numpy as jnp
from jax import lax
from jax.experimental import pallas as pl
from jax.experimental.pallas import tpu as pltpu

N_DEV = 8
B = 2
SQ = 128
D = 512
HL = 8
DH = 64


def kernel(x, Wq, Wo, K_ext, V_ext):
    my = lax.axis_index("i")
    bf = jnp.bfloat16
    K_loc = lax.dynamic_slice(K_ext, (0, 0, my * HL, 0), (B, SQ, HL, DH))
    V_loc = lax.dynamic_slice(V_ext, (0, 0, my * HL, 0), (B, SQ, HL, DH))
    K_loc = jnp.transpose(K_loc.astype(bf), (0, 2, 1, 3)).reshape(
        B * HL, SQ, DH)
    V_loc = jnp.transpose(V_loc.astype(bf), (0, 2, 1, 3)).reshape(
        B * HL, SQ, DH)
    x = x.astype(bf)
    Wq = Wq.astype(bf)
    Wo = Wo.astype(bf)

    def body(x_ref, wq_ref, wo_ref, k_ref, v_ref, out_ref,
             xA0, xA1, xB0, xB1, aA0, aA1, aB0, aB1, zx, homes, q_in, o_scr,
             xA0s, xA0r, xA1s, xA1r, xB0s, xB0r, xB1s, xB1r,
             aA0s, aA0r, aA1s, aA1r, aB0s, aB0r, aB1s, aB1r,
             zs, zr, hs, hr):
        me = lax.axis_index("i")
        right = lax.rem(me + 1, N_DEV)
        left = lax.rem(me + N_DEV - 1, N_DEV)
        twin = lax.rem(me + 4, N_DEV)
        fwd3 = lax.rem(me + 3, N_DEV)
        back3 = lax.rem(me + N_DEV - 3, N_DEV)

        barrier_sem = pltpu.get_barrier_semaphore()
        for nbr in (left, right, twin, fwd3, back3):
            pl.semaphore_signal(
                barrier_sem, inc=1,
                device_id=(nbr,), device_id_type=pl.DeviceIdType.MESH,
            )
        pl.semaphore_wait(barrier_sem, 5)

        def marc(src, dst, ssem, rsem, dev):
            return pltpu.make_async_remote_copy(
                src_ref=src, dst_ref=dst, send_sem=ssem, recv_sem=rsem,
                device_id=(dev,), device_id_type=pl.DeviceIdType.MESH,
            )

        def m_xA0(i):
            return marc(xA0.at[i], xA0.at[i + 1], xA0s.at[i], xA0r.at[i],
                        right)

        def m_xA1(i):
            return marc(xA1.at[i], xA1.at[i + 1], xA1s.at[i], xA1r.at[i],
                        left)

        def m_xB0(i, seed=False):
            src = zx.at[0] if seed else xB0.at[i]
            return marc(src, xB0.at[i + 1], xB0s.at[i], xB0r.at[i], right)

        def m_xB1(i, seed=False):
            src = zx.at[1] if seed else xB1.at[i]
            return marc(src, xB1.at[i + 1], xB1s.at[i], xB1r.at[i], left)

        def m_aA0(i):
            return marc(aA0.at[i], aA0.at[i + 1], aA0s.at[i], aA0r.at[i],
                        right)

        def m_aA1(i):
            return marc(aA1.at[i], aA1.at[i + 1], aA1s.at[i], aA1r.at[i],
                        left)

        def m_aB0(i):
            return marc(aB0.at[i], aB0.at[i + 1], aB0s.at[i], aB0r.at[i],
                        right)

        def m_aB1(i):
            return marc(aB1.at[i], aB1.at[i + 1], aB1s.at[i], aB1r.at[i],
                        left)

        m_z = marc(x_ref, zx, zs, zr, twin)
        m_hA0 = marc(aA0.at[3], homes.at[0], hs.at[0], hr.at[0], back3)
        m_hB0 = marc(aB0.at[3], homes.at[1], hs.at[1], hr.at[1], right)
        m_hA1 = marc(aA1.at[3], homes.at[2], hs.at[2], hr.at[2], fwd3)
        m_hB1 = marc(aB1.at[3], homes.at[3], hs.at[3], hr.at[3], left)

        def contrib_pair(xR, xL):
            q_in[0:SQ, :] = xR
            q_in[SQ:2 * SQ, :] = xL
            q = jnp.dot(q_in[...], wq_ref[...],
                        preferred_element_type=jnp.float32
                        ).astype(jnp.bfloat16)
            q4 = q.reshape(B, SQ, HL, DH)
            k4 = k_ref[...].reshape(B, HL, SQ, DH)
            v4 = v_ref[...].reshape(B, HL, SQ, DH)
            for h in range(HL):
                qh = q4[:, :, h, :]
                s = lax.dot_general(
                    qh, k4[:, h], (((2,), (2,)), ((0,), (0,))),
                    preferred_element_type=jnp.float32) * 0.125
                m = jnp.max(s, axis=2, keepdims=True)
                p = jnp.exp(s - m)
                l = jnp.sum(p, axis=2, keepdims=True)
                o = lax.dot_general(
                    p.astype(jnp.bfloat16), v4[:, h],
                    (((2,), (1,)), ((0,), (0,))),
                    preferred_element_type=jnp.float32) / l
                o_scr[:, :, h * DH:(h + 1) * DH] = o.astype(jnp.bfloat16)
            o2 = o_scr[...].reshape(B * SQ, HL * DH)
            return jnp.dot(o2, wo_ref[...],
                           preferred_element_type=jnp.float32
                           ).reshape(B, SQ, D)

        xA0[0] = x_ref[0]
        xA1[0] = x_ref[1]
        m_z.start()
        m_xA0(0).start()
        m_xA1(0).start()
        c = contrib_pair(x_ref[0], x_ref[1]).astype(jnp.bfloat16)
        aA0[0] = c[0]
        aA1[0] = c[1]
        m_aA0(0).start()
        m_aA1(0).start()

        m_z.wait_recv()
        m_xB0(0, seed=True).start()
        m_xB1(0, seed=True).start()
        c = contrib_pair(zx[0], zx[1]).astype(jnp.bfloat16)
        aB0[0] = c[0]
        aB1[0] = c[1]
        m_aB0(0).start()
        m_aB1(0).start()

        def relay(h, _):
            m_xA0(h - 1).wait_recv()
            m_xA0(h).start()
            m_xA1(h - 1).wait_recv()
            m_xA1(h).start()
            c = contrib_pair(xA0.at[h][...], xA1.at[h][...]
                             ).astype(jnp.bfloat16)
            m_aA0(h - 1).wait_recv()
            aref = aA0.at[h]
            aref[...] = aref[...] + c[0]
            m_aA0(h).start()
            m_aA1(h - 1).wait_recv()
            aref = aA1.at[h]
            aref[...] = aref[...] + c[1]
            m_aA1(h).start()

            m_xB0(h - 1).wait_recv()
            m_xB0(h).start()
            m_xB1(h - 1).wait_recv()
            m_xB1(h).start()
            c = contrib_pair(xB0.at[h][...], xB1.at[h][...]
                             ).astype(jnp.bfloat16)
            m_aB0(h - 1).wait_recv()
            aref = aB0.at[h]
            aref[...] = aref[...] + c[0]
            m_aB0(h).start()
            m_aB1(h - 1).wait_recv()
            aref = aB1.at[h]
            aref[...] = aref[...] + c[1]
            m_aB1(h).start()
            return _

        lax.fori_loop(1, 3, relay, None)

        m_xA0(2).wait_recv()
        m_xA1(2).wait_recv()
        c = contrib_pair(xA0.at[3][...], xA1.at[3][...]).astype(jnp.bfloat16)
        m_aA0(2).wait_recv()
        aref = aA0.at[3]
        aref[...] = aref[...] + c[0]
        m_hA0.start()
        m_aA1(2).wait_recv()
        aref = aA1.at[3]
        aref[...] = aref[...] + c[1]
        m_hA1.start()

        m_xB0(2).wait_recv()
        m_xB1(2).wait_recv()
        c = contrib_pair(xB0.at[3][...], xB1.at[3][...]).astype(jnp.bfloat16)
        m_aB0(2).wait_recv()
        aref = aB0.at[3]
        aref[...] = aref[...] + c[0]
        m_hB0.start()
        m_aB1(2).wait_recv()
        aref = aB1.at[3]
        aref[...] = aref[...] + c[1]
        m_hB1.start()

        m_hA0.wait_recv()
        m_hB0.wait_recv()
        out_ref[0] = homes[0] + homes[1]
        m_hA1.wait_recv()
        m_hB1.wait_recv()
        out_ref[1] = homes[2] + homes[3]

        def drain(h, _):
            m_xA0(h).wait_send()
            m_xA1(h).wait_send()
            m_xB0(h).wait_send()
            m_xB1(h).wait_send()
            m_aA0(h).wait_send()
            m_aA1(h).wait_send()
            m_aB0(h).wait_send()
            m_aB1(h).wait_send()
            return _

        lax.fori_loop(0, 3, drain, None)
        m_z.wait_send()
        m_hA0.wait_send()
        m_hB0.wait_send()
        m_hA1.wait_send()
        m_hB1.wait_send()

    dma = pltpu.SemaphoreType.DMA
    out = pl.pallas_call(
        body,
        out_shape=jax.ShapeDtypeStruct((B, SQ, D), jnp.bfloat16),
        in_specs=[pl.BlockSpec(memory_space=pltpu.VMEM)] * 5,
        out_specs=pl.BlockSpec(memory_space=pltpu.VMEM),
        scratch_shapes=[
            pltpu.VMEM((4, SQ, D), jnp.bfloat16),
            pltpu.VMEM((4, SQ, D), jnp.bfloat16),
            pltpu.VMEM((4, SQ, D), jnp.bfloat16),
            pltpu.VMEM((4, SQ, D), jnp.bfloat16),
            pltpu.VMEM((4, SQ, D), jnp.bfloat16),
            pltpu.VMEM((4, SQ, D), jnp.bfloat16),
            pltpu.VMEM((4, SQ, D), jnp.bfloat16),
            pltpu.VMEM((4, SQ, D), jnp.bfloat16),
            pltpu.VMEM((B, SQ, D), jnp.bfloat16),
            pltpu.VMEM((4, SQ, D), jnp.bfloat16),
            pltpu.VMEM((B * SQ, D), jnp.bfloat16),
            pltpu.VMEM((B, SQ, HL * DH), jnp.bfloat16),
            dma((4,)), dma((4,)),
            dma((4,)), dma((4,)),
            dma((4,)), dma((4,)),
            dma((4,)), dma((4,)),
            dma((4,)), dma((4,)),
            dma((4,)), dma((4,)),
            dma((4,)), dma((4,)),
            dma((4,)), dma((4,)),
            dma(()), dma(()),
            dma((4,)), dma((4,)),
        ],
        compiler_params=pltpu.CompilerParams(collective_id=0),
    )(x, Wq, Wo, K_loc, V_loc)

    return out.astype(jnp.float32)


# device time: 37619 ns/iter; 1.1549x vs baseline; 1.1549x over previous
import jax
import jax.numpy as jnp
from jax import lax
from jax.experimental import pallas as pl
from jax.experimental.pallas import tpu as pltpu

N_DEV = 8
B = 2
SQ = 128
D = 512
HL = 8
DH = 64


def kernel(x, Wq, Wo, K_ext, V_ext):
    my = lax.axis_index("i")
    bf = jnp.bfloat16
    K_loc = lax.dynamic_slice(K_ext, (0, 0, my * HL, 0), (B, SQ, HL, DH))
    V_loc = lax.dynamic_slice(V_ext, (0, 0, my * HL, 0), (B, SQ, HL, DH))
    K_loc = jnp.transpose(K_loc.astype(bf), (0, 2, 1, 3)).reshape(
        B * HL, SQ, DH)
    V_loc = jnp.transpose(V_loc.astype(bf), (0, 2, 1, 3)).reshape(
        B * HL, SQ, DH)
    x = x.astype(bf)
    Wq = Wq.astype(bf)
    Wo = Wo.astype(bf)

    def body(x_ref, wq_ref, wo_ref, k_ref, v_ref, out_ref,
             xA0, xA1, xB0, xB1, aA0, aA1, aB0, aB1, zx, homes, q_in, o_scr,
             xA0s, xA0r, xA1s, xA1r, xB0s, xB0r, xB1s, xB1r,
             aA0s, aA0r, aA1s, aA1r, aB0s, aB0r, aB1s, aB1r,
             zs, zr, hs, hr):
        me = lax.axis_index("i")
        right = lax.rem(me + 1, N_DEV)
        left = lax.rem(me + N_DEV - 1, N_DEV)
        twin = lax.rem(me + 4, N_DEV)
        fwd3 = lax.rem(me + 3, N_DEV)
        back3 = lax.rem(me + N_DEV - 3, N_DEV)

        barrier_sem = pltpu.get_barrier_semaphore()
        for nbr in (left, right, twin, fwd3, back3):
            pl.semaphore_signal(
                barrier_sem, inc=1,
                device_id=(nbr,), device_id_type=pl.DeviceIdType.MESH,
            )
        pl.semaphore_wait(barrier_sem, 5)

        def marc(src, dst, ssem, rsem, dev):
            return pltpu.make_async_remote_copy(
                src_ref=src, dst_ref=dst, send_sem=ssem, recv_sem=rsem,
                device_id=(dev,), device_id_type=pl.DeviceIdType.MESH,
            )

        def m_xA0(i):
            return marc(xA0.at[i], xA0.at[i + 1], xA0s.at[i], xA0r.at[i],
                        right)

        def m_xA1(i):
            return marc(xA1.at[i], xA1.at[i + 1], xA1s.at[i], xA1r.at[i],
                        left)

        def m_xB0(i, seed=False):
            src = zx.at[0] if seed else xB0.at[i]
            return marc(src, xB0.at[i + 1], xB0s.at[i], xB0r.at[i], right)

        def m_xB1(i, seed=False):
            src = zx.at[1] if seed else xB1.at[i]
            return marc(src, xB1.at[i + 1], xB1s.at[i], xB1r.at[i], left)

        def m_aA0(i):
            return marc(aA0.at[i], aA0.at[i + 1], aA0s.at[i], aA0r.at[i],
                        right)

        def m_aA1(i):
            return marc(aA1.at[i], aA1.at[i + 1], aA1s.at[i], aA1r.at[i],
                        left)

        def m_aB0(i):
            return marc(aB0.at[i], aB0.at[i + 1], aB0s.at[i], aB0r.at[i],
                        right)

        def m_aB1(i):
            return marc(aB1.at[i], aB1.at[i + 1], aB1s.at[i], aB1r.at[i],
                        left)

        m_z = marc(x_ref, zx, zs, zr, twin)
        m_hA0 = marc(aA0.at[3], homes.at[0], hs.at[0], hr.at[0], back3)
        m_hB0 = marc(aB0.at[3], homes.at[1], hs.at[1], hr.at[1], right)
        m_hA1 = marc(aA1.at[3], homes.at[2], hs.at[2], hr.at[2], fwd3)
        m_hB1 = marc(aB1.at[3], homes.at[3], hs.at[3], hr.at[3], left)

        def contrib_pair(xR, xL):
            q_in[0:SQ, :] = xR
            q_in[SQ:2 * SQ, :] = xL
            q = jnp.dot(q_in[...], wq_ref[...],
                        preferred_element_type=jnp.float32
                        ).astype(jnp.bfloat16)
            q3 = jnp.transpose(q.reshape(B, SQ, HL, DH), (0, 2, 1, 3)
                               ).reshape(B * HL, SQ, DH)
            s = lax.dot_general(
                q3, k_ref[...], (((2,), (2,)), ((0,), (0,))),
                preferred_element_type=jnp.float32) * 0.125
            m = jnp.max(s, axis=2, keepdims=True)
            p = jnp.exp(s - m)
            l = jnp.sum(p, axis=2, keepdims=True)
            o = lax.dot_general(
                p.astype(jnp.bfloat16), v_ref[...],
                (((2,), (1,)), ((0,), (0,))),
                preferred_element_type=jnp.float32) / l
            o2 = jnp.transpose(
                o.astype(jnp.bfloat16).reshape(B, HL, SQ, DH), (0, 2, 1, 3)
            ).reshape(B * SQ, HL * DH)
            return jnp.dot(o2, wo_ref[...],
                           preferred_element_type=jnp.float32
                           ).reshape(B, SQ, D)

        xA0[0] = x_ref[0]
        xA1[0] = x_ref[1]
        m_z.start()
        m_xA0(0).start()
        m_xA1(0).start()
        c = contrib_pair(x_ref[0], x_ref[1]).astype(jnp.bfloat16)
        aA0[0] = c[0]
        aA1[0] = c[1]
        m_aA0(0).start()
        m_aA1(0).start()

        m_z.wait_recv()
        m_xB0(0, seed=True).start()
        m_xB1(0, seed=True).start()
        c = contrib_pair(zx[0], zx[1]).astype(jnp.bfloat16)
        aB0[0] = c[0]
        aB1[0] = c[1]
        m_aB0(0).start()
        m_aB1(0).start()

        def relay(h, _):
            m_xA0(h - 1).wait_recv()
            m_xA0(h).start()
            m_xA1(h - 1).wait_recv()
            m_xA1(h).start()
            c = contrib_pair(xA0.at[h][...], xA1.at[h][...]
                             ).astype(jnp.bfloat16)
            m_aA0(h - 1).wait_recv()
            aref = aA0.at[h]
            aref[...] = aref[...] + c[0]
            m_aA0(h).start()
            m_aA1(h - 1).wait_recv()
            aref = aA1.at[h]
            aref[...] = aref[...] + c[1]
            m_aA1(h).start()

            m_xB0(h - 1).wait_recv()
            m_xB0(h).start()
            m_xB1(h - 1).wait_recv()
            m_xB1(h).start()
            c = contrib_pair(xB0.at[h][...], xB1.at[h][...]
                             ).astype(jnp.bfloat16)
            m_aB0(h - 1).wait_recv()
            aref = aB0.at[h]
            aref[...] = aref[...] + c[0]
            m_aB0(h).start()
            m_aB1(h - 1).wait_recv()
            aref = aB1.at[h]
            aref[...] = aref[...] + c[1]
            m_aB1(h).start()
            return _

        lax.fori_loop(1, 3, relay, None)

        m_xA0(2).wait_recv()
        m_xA1(2).wait_recv()
        c = contrib_pair(xA0.at[3][...], xA1.at[3][...]).astype(jnp.bfloat16)
        m_aA0(2).wait_recv()
        aref = aA0.at[3]
        aref[...] = aref[...] + c[0]
        m_hA0.start()
        m_aA1(2).wait_recv()
        aref = aA1.at[3]
        aref[...] = aref[...] + c[1]
        m_hA1.start()

        m_xB0(2).wait_recv()
        m_xB1(2).wait_recv()
        c = contrib_pair(xB0.at[3][...], xB1.at[3][...]).astype(jnp.bfloat16)
        m_aB0(2).wait_recv()
        aref = aB0.at[3]
        aref[...] = aref[...] + c[0]
        m_hB0.start()
        m_aB1(2).wait_recv()
        aref = aB1.at[3]
        aref[...] = aref[...] + c[1]
        m_hB1.start()

        m_hA0.wait_recv()
        m_hB0.wait_recv()
        out_ref[0] = homes[0] + homes[1]
        m_hA1.wait_recv()
        m_hB1.wait_recv()
        out_ref[1] = homes[2] + homes[3]

        def drain(h, _):
            m_xA0(h).wait_send()
            m_xA1(h).wait_send()
            m_xB0(h).wait_send()
            m_xB1(h).wait_send()
            m_aA0(h).wait_send()
            m_aA1(h).wait_send()
            m_aB0(h).wait_send()
            m_aB1(h).wait_send()
            return _

        lax.fori_loop(0, 3, drain, None)
        m_z.wait_send()
        m_hA0.wait_send()
        m_hB0.wait_send()
        m_hA1.wait_send()
        m_hB1.wait_send()

    dma = pltpu.SemaphoreType.DMA
    out = pl.pallas_call(
        body,
        out_shape=jax.ShapeDtypeStruct((B, SQ, D), jnp.bfloat16),
        in_specs=[pl.BlockSpec(memory_space=pltpu.VMEM)] * 5,
        out_specs=pl.BlockSpec(memory_space=pltpu.VMEM),
        scratch_shapes=[
            pltpu.VMEM((4, SQ, D), jnp.bfloat16),
            pltpu.VMEM((4, SQ, D), jnp.bfloat16),
            pltpu.VMEM((4, SQ, D), jnp.bfloat16),
            pltpu.VMEM((4, SQ, D), jnp.bfloat16),
            pltpu.VMEM((4, SQ, D), jnp.bfloat16),
            pltpu.VMEM((4, SQ, D), jnp.bfloat16),
            pltpu.VMEM((4, SQ, D), jnp.bfloat16),
            pltpu.VMEM((4, SQ, D), jnp.bfloat16),
            pltpu.VMEM((B, SQ, D), jnp.bfloat16),
            pltpu.VMEM((4, SQ, D), jnp.bfloat16),
            pltpu.VMEM((B * SQ, D), jnp.bfloat16),
            pltpu.VMEM((B, SQ, HL * DH), jnp.bfloat16),
            dma((4,)), dma((4,)),
            dma((4,)), dma((4,)),
            dma((4,)), dma((4,)),
            dma((4,)), dma((4,)),
            dma((4,)), dma((4,)),
            dma((4,)), dma((4,)),
            dma((4,)), dma((4,)),
            dma((4,)), dma((4,)),
            dma(()), dma(()),
            dma((4,)), dma((4,)),
        ],
        compiler_params=pltpu.CompilerParams(collective_id=0),
    )(x, Wq, Wo, K_loc, V_loc)

    return out.astype(jnp.float32)


# device time: 37554 ns/iter; 1.1569x vs baseline; 1.0017x over previous
import jax
import jax.numpy as jnp
from jax import lax
from jax.experimental import pallas as pl
from jax.experimental.pallas import tpu as pltpu

N_DEV = 8
B = 2
SQ = 128
D = 512
HL = 8
DH = 64


def kernel(x, Wq, Wo, K_ext, V_ext):
    my = lax.axis_index("i")
    bf = jnp.bfloat16
    K_loc = lax.dynamic_slice(K_ext, (0, 0, my * HL, 0), (B, SQ, HL, DH))
    V_loc = lax.dynamic_slice(V_ext, (0, 0, my * HL, 0), (B, SQ, HL, DH))
    K_loc = jnp.transpose(K_loc.astype(bf), (0, 2, 1, 3)).reshape(
        B * HL, SQ, DH)
    V_loc = jnp.transpose(V_loc.astype(bf), (0, 2, 1, 3)).reshape(
        B * HL, SQ, DH)
    x = x.astype(bf)
    Wq = (Wq * 0.125).astype(bf)
    Wo = Wo.astype(bf)

    def body(x_ref, wq_ref, wo_ref, k_ref, v_ref, out_ref,
             xA0, xA1, xB0, xB1, aA0, aA1, aB0, aB1, zx, homes, q_in, o_scr,
             xA0s, xA0r, xA1s, xA1r, xB0s, xB0r, xB1s, xB1r,
             aA0s, aA0r, aA1s, aA1r, aB0s, aB0r, aB1s, aB1r,
             zs, zr, hs, hr):
        me = lax.axis_index("i")
        right = lax.rem(me + 1, N_DEV)
        left = lax.rem(me + N_DEV - 1, N_DEV)
        twin = lax.rem(me + 4, N_DEV)
        fwd3 = lax.rem(me + 3, N_DEV)
        back3 = lax.rem(me + N_DEV - 3, N_DEV)

        barrier_sem = pltpu.get_barrier_semaphore()
        for nbr in (left, right, twin, fwd3, back3):
            pl.semaphore_signal(
                barrier_sem, inc=1,
                device_id=(nbr,), device_id_type=pl.DeviceIdType.MESH,
            )
        pl.semaphore_wait(barrier_sem, 5)

        def marc(src, dst, ssem, rsem, dev):
            return pltpu.make_async_remote_copy(
                src_ref=src, dst_ref=dst, send_sem=ssem, recv_sem=rsem,
                device_id=(dev,), device_id_type=pl.DeviceIdType.MESH,
            )

        def m_xA0(i):
            return marc(xA0.at[i], xA0.at[i + 1], xA0s.at[i], xA0r.at[i],
                        right)

        def m_xA1(i):
            return marc(xA1.at[i], xA1.at[i + 1], xA1s.at[i], xA1r.at[i],
                        left)

        def m_xB0(i, seed=False):
            src = zx.at[0] if seed else xB0.at[i]
            return marc(src, xB0.at[i + 1], xB0s.at[i], xB0r.at[i], right)

        def m_xB1(i, seed=False):
            src = zx.at[1] if seed else xB1.at[i]
            return marc(src, xB1.at[i + 1], xB1s.at[i], xB1r.at[i], left)

        def m_aA0(i):
            return marc(aA0.at[i], aA0.at[i + 1], aA0s.at[i], aA0r.at[i],
                        right)

        def m_aA1(i):
            return marc(aA1.at[i], aA1.at[i + 1], aA1s.at[i], aA1r.at[i],
                        left)

        def m_aB0(i):
            return marc(aB0.at[i], aB0.at[i + 1], aB0s.at[i], aB0r.at[i],
                        right)

        def m_aB1(i):
            return marc(aB1.at[i], aB1.at[i + 1], aB1s.at[i], aB1r.at[i],
                        left)

        m_z = marc(x_ref, zx, zs, zr, twin)
        m_hA0 = marc(aA0.at[3], homes.at[0], hs.at[0], hr.at[0], back3)
        m_hB0 = marc(aB0.at[3], homes.at[1], hs.at[1], hr.at[1], right)
        m_hA1 = marc(aA1.at[3], homes.at[2], hs.at[2], hr.at[2], fwd3)
        m_hB1 = marc(aB1.at[3], homes.at[3], hs.at[3], hr.at[3], left)

        def contrib_pair(xR, xL):
            q_in[0:SQ, :] = xR
            q_in[SQ:2 * SQ, :] = xL
            q = jnp.dot(q_in[...], wq_ref[...],
                        preferred_element_type=jnp.float32
                        ).astype(jnp.bfloat16)
            q3 = jnp.transpose(q.reshape(B, SQ, HL, DH), (0, 2, 1, 3)
                               ).reshape(B * HL, SQ, DH)
            s = lax.dot_general(
                q3, k_ref[...], (((2,), (2,)), ((0,), (0,))),
                preferred_element_type=jnp.float32)
            p = jnp.exp(s)
            l = jnp.sum(p, axis=2, keepdims=True)
            o = lax.dot_general(
                p.astype(jnp.bfloat16), v_ref[...],
                (((2,), (1,)), ((0,), (0,))),
                preferred_element_type=jnp.float32) * (1.0 / l)
            o2 = jnp.transpose(
                o.astype(jnp.bfloat16).reshape(B, HL, SQ, DH), (0, 2, 1, 3)
            ).reshape(B * SQ, HL * DH)
            return jnp.dot(o2, wo_ref[...],
                           preferred_element_type=jnp.float32
                           ).reshape(B, SQ, D)

        xA0[0] = x_ref[0]
        xA1[0] = x_ref[1]
        m_z.start()
        m_xA0(0).start()
        m_xA1(0).start()
        c = contrib_pair(x_ref[0], x_ref[1]).astype(jnp.bfloat16)
        aA0[0] = c[0]
        aA1[0] = c[1]
        m_aA0(0).start()
        m_aA1(0).start()

        m_z.wait_recv()
        m_xB0(0, seed=True).start()
        m_xB1(0, seed=True).start()
        c = contrib_pair(zx[0], zx[1]).astype(jnp.bfloat16)
        aB0[0] = c[0]
        aB1[0] = c[1]
        m_aB0(0).start()
        m_aB1(0).start()

        def relay(h, _):
            m_xA0(h - 1).wait_recv()
            m_xA0(h).start()
            m_xA1(h - 1).wait_recv()
            m_xA1(h).start()
            c = contrib_pair(xA0.at[h][...], xA1.at[h][...]
                             ).astype(jnp.bfloat16)
            m_aA0(h - 1).wait_recv()
            aref = aA0.at[h]
            aref[...] = aref[...] + c[0]
            m_aA0(h).start()
            m_aA1(h - 1).wait_recv()
            aref = aA1.at[h]
            aref[...] = aref[...] + c[1]
            m_aA1(h).start()

            m_xB0(h - 1).wait_recv()
            m_xB0(h).start()
            m_xB1(h - 1).wait_recv()
            m_xB1(h).start()
            c = contrib_pair(xB0.at[h][...], xB1.at[h][...]
                             ).astype(jnp.bfloat16)
            m_aB0(h - 1).wait_recv()
            aref = aB0.at[h]
            aref[...] = aref[...] + c[0]
            m_aB0(h).start()
            m_aB1(h - 1).wait_recv()
            aref = aB1.at[h]
            aref[...] = aref[...] + c[1]
            m_aB1(h).start()
            return _

        lax.fori_loop(1, 3, relay, None)

        m_xA0(2).wait_recv()
        m_xA1(2).wait_recv()
        c = contrib_pair(xA0.at[3][...], xA1.at[3][...]).astype(jnp.bfloat16)
        m_aA0(2).wait_recv()
        aref = aA0.at[3]
        aref[...] = aref[...] + c[0]
        m_hA0.start()
        m_aA1(2).wait_recv()
        aref = aA1.at[3]
        aref[...] = aref[...] + c[1]
        m_hA1.start()

        m_xB0(2).wait_recv()
        m_xB1(2).wait_recv()
        c = contrib_pair(xB0.at[3][...], xB1.at[3][...]).astype(jnp.bfloat16)
        m_aB0(2).wait_recv()
        aref = aB0.at[3]
        aref[...] = aref[...] + c[0]
        m_hB0.start()
        m_aB1(2).wait_recv()
        aref = aB1.at[3]
        aref[...] = aref[...] + c[1]
        m_hB1.start()

        m_hA0.wait_recv()
        m_hB0.wait_recv()
        out_ref[0] = homes[0] + homes[1]
        m_hA1.wait_recv()
        m_hB1.wait_recv()
        out_ref[1] = homes[2] + homes[3]

        def drain(h, _):
            m_xA0(h).wait_send()
            m_xA1(h).wait_send()
            m_xB0(h).wait_send()
            m_xB1(h).wait_send()
            m_aA0(h).wait_send()
            m_aA1(h).wait_send()
            m_aB0(h).wait_send()
            m_aB1(h).wait_send()
            return _

        lax.fori_loop(0, 3, drain, None)
        m_z.wait_send()
        m_hA0.wait_send()
        m_hB0.wait_send()
        m_hA1.wait_send()
        m_hB1.wait_send()

    dma = pltpu.SemaphoreType.DMA
    out = pl.pallas_call(
        body,
        out_shape=jax.ShapeDtypeStruct((B, SQ, D), jnp.bfloat16),
        in_specs=[pl.BlockSpec(memory_space=pltpu.VMEM)] * 5,
        out_specs=pl.BlockSpec(memory_space=pltpu.VMEM),
        scratch_shapes=[
            pltpu.VMEM((4, SQ, D), jnp.bfloat16),
            pltpu.VMEM((4, SQ, D), jnp.bfloat16),
            pltpu.VMEM((4, SQ, D), jnp.bfloat16),
            pltpu.VMEM((4, SQ, D), jnp.bfloat16),
            pltpu.VMEM((4, SQ, D), jnp.bfloat16),
            pltpu.VMEM((4, SQ, D), jnp.bfloat16),
            pltpu.VMEM((4, SQ, D), jnp.bfloat16),
            pltpu.VMEM((4, SQ, D), jnp.bfloat16),
            pltpu.VMEM((B, SQ, D), jnp.bfloat16),
            pltpu.VMEM((4, SQ, D), jnp.bfloat16),
            pltpu.VMEM((B * SQ, D), jnp.bfloat16),
            pltpu.VMEM((B, SQ, HL * DH), jnp.bfloat16),
            dma((4,)), dma((4,)),
            dma((4,)), dma((4,)),
            dma((4,)), dma((4,)),
            dma((4,)), dma((4,)),
            dma((4,)), dma((4,)),
            dma((4,)), dma((4,)),
            dma((4,)), dma((4,)),
            dma((4,)), dma((4,)),
            dma(()), dma(()),
            dma((4,)), dma((4,)),
        ],
        compiler_params=pltpu.CompilerParams(collective_id=0),
    )(x, Wq, Wo, K_loc, V_loc)

    return out.astype(jnp.float32)
